# baseline (speedup 1.0000x reference)
import functools

import jax
import jax.numpy as jnp
from jax import lax
from jax.experimental import pallas as pl
from jax.experimental.pallas import tpu as pltpu
from jax.experimental.pallas import tpu_sc as plsc

_VOCAB = 1000000
_EMBED = 32
_BATCH = 16384
_HIST = 200

_NC = 2
_NS = 16
_NW = _NC * _NS
_RPW = _BATCH // _NW
_RC = 8
_NSTEPS = _RPW // _RC


def _gather_body(idx_hbm, table_hbm, out_hbm, idx_v, rows_v, gsem, osem0,
                 osem1):
  wid = lax.axis_index("s") * _NC + lax.axis_index("c")
  base = wid * _RPW
  osems = (osem0, osem1)

  def do_step(i, b):
    row0 = base + i * _RC
    pltpu.sync_copy(idx_hbm.at[pl.ds(row0, _RC), :], idx_v.at[b])

    @pl.when(i >= 2)
    def _():
      def dr(k, carry):
        pltpu.make_async_copy(
            rows_v.at[b, :, pl.ds(4 * k, 4), :],
            out_hbm.at[0, pl.ds(row0, _RC)], osems[b]).wait()
        return carry
      lax.fori_loop(0, 50, dr, 0)

    for j in range(_RC):
      pltpu.async_copy(table_hbm.at[idx_v.at[b, j]], rows_v.at[b, j], gsem)
    for j in range(_RC):
      pltpu.make_async_copy(
          table_hbm.at[idx_v.at[b, j]], rows_v.at[b, j], gsem).wait()

    def wr(k, carry):
      pltpu.async_copy(
          rows_v.at[b, :, pl.ds(4 * k, 4), :],
          out_hbm.at[k, pl.ds(row0, _RC)], osems[b])
      return carry
    lax.fori_loop(0, 50, wr, 0)

  def pair(g, carry):
    do_step(g * 2, 0)
    do_step(g * 2 + 1, 1)
    return carry

  lax.fori_loop(0, _NSTEPS // 2, pair, 0)
  def drain_all(k, carry):
    pltpu.make_async_copy(
        rows_v.at[0, :, pl.ds(4 * k, 4), :],
        out_hbm.at[0, pl.ds(base, _RC)], osem0).wait()
    return carry
  lax.fori_loop(0, 50, drain_all, 0)
  def drain_all2(k, carry):
    pltpu.make_async_copy(
        rows_v.at[1, :, pl.ds(4 * k, 4), :],
        out_hbm.at[0, pl.ds(base, _RC)], osem1).wait()
    return carry
  lax.fori_loop(0, 50, drain_all2, 0)


_gather = functools.partial(
    pl.kernel,
    out_type=jax.ShapeDtypeStruct((50, _BATCH, 4, _EMBED), jnp.float32),
    mesh=plsc.VectorSubcoreMesh(core_axis_name="c", subcore_axis_name="s"),
    scratch_types=[
        pltpu.VMEM((2, _RC, _HIST), jnp.int32),
        pltpu.VMEM((2, _RC, _HIST, _EMBED), jnp.float32),
        pltpu.SemaphoreType.DMA,
        pltpu.SemaphoreType.DMA,
        pltpu.SemaphoreType.DMA,
    ],
    compiler_params=pltpu.CompilerParams(use_tc_tiling_on_sc=False),
)(_gather_body)


def _retile_body(x_ref, o_ref):
  for ct in range(16):
    xc = x_ref[:, ct * 8:(ct + 1) * 8]
    x3 = jnp.reshape(xc, (16, 128, 8))
    o_ref[0, ct] = jnp.transpose(x3, (0, 2, 1))


_retile = pl.pallas_call(
    _retile_body,
    out_shape=jax.ShapeDtypeStruct((50, 16, 128, 8, 128), jnp.float32),
    grid=(50, 8),
    in_specs=[pl.BlockSpec((2048, 128), lambda k, bb: (k * 8 + bb, 0))],
    out_specs=pl.BlockSpec(
        (1, 16, 16, 8, 128), lambda k, bb: (k, 0, bb, 0, 0)),
)


def _mask_body(x_ref, o_ref):
  o_ref[...] = x_ref[...] != 0


_mask = pl.pallas_call(
    _mask_body,
    out_shape=jax.ShapeDtypeStruct((_BATCH, _HIST), jnp.bool_),
    grid=(16,),
    in_specs=[pl.BlockSpec((_BATCH // 16, _HIST), lambda i: (i, 0))],
    out_specs=pl.BlockSpec((_BATCH // 16, _HIST), lambda i: (i, 0)),
)


@jax.jit
def kernel(inputs, table):
  rows4 = _gather(inputs, table)
  mask = _mask(inputs)
  out5 = _retile(jnp.reshape(rows4, (50 * _BATCH, 128)))
  emb = jnp.reshape(
      jnp.transpose(out5, (2, 4, 0, 1, 3)), (_BATCH, _HIST, _EMBED))
  return emb, mask


# speedup vs baseline: 3.1431x; 3.1431x over previous
import functools

import jax
import jax.numpy as jnp
from jax import lax
from jax.experimental import pallas as pl
from jax.experimental.pallas import tpu as pltpu
from jax.experimental.pallas import tpu_sc as plsc

_VOCAB = 1000000
_EMBED = 32
_BATCH = 16384
_HIST = 200

_NC = 2
_NS = 16
_NW = _NC * _NS
_RPW = _BATCH // _NW
_RC = 8
_NSTEPS = _RPW // _RC


def _gather_body(idx_hbm, table_hbm, out_hbm, idx_v, rows_v, gsem, osem0,
                 osem1):
  wid = lax.axis_index("s") * _NC + lax.axis_index("c")
  base = wid * _RPW
  osems = (osem0, osem1)

  def do_step(i, b):
    row0 = base + i * _RC
    pltpu.sync_copy(idx_hbm.at[pl.ds(row0, _RC), :], idx_v.at[b])

    @pl.when(i >= 2)
    def _():
      def dr(k, carry):
        pltpu.make_async_copy(
            rows_v.at[b, :, pl.ds(4 * k, 4), :],
            out_hbm.at[0, pl.ds(row0, _RC)], osems[b]).wait()
        return carry
      lax.fori_loop(0, 50, dr, 0)

    for j in range(_RC):
      pltpu.async_copy(table_hbm.at[idx_v.at[b, j]], rows_v.at[b, j], gsem)
    for j in range(_RC):
      pltpu.make_async_copy(
          table_hbm.at[idx_v.at[b, j]], rows_v.at[b, j], gsem).wait()

    def wr(k, carry):
      pltpu.async_copy(
          rows_v.at[b, :, pl.ds(4 * k, 4), :],
          out_hbm.at[k, pl.ds(row0, _RC)], osems[b])
      return carry
    lax.fori_loop(0, 50, wr, 0)

  def pair(g, carry):
    do_step(g * 2, 0)
    do_step(g * 2 + 1, 1)
    return carry

  lax.fori_loop(0, _NSTEPS // 2, pair, 0)
  def drain_all(k, carry):
    pltpu.make_async_copy(
        rows_v.at[0, :, pl.ds(4 * k, 4), :],
        out_hbm.at[0, pl.ds(base, _RC)], osem0).wait()
    return carry
  lax.fori_loop(0, 50, drain_all, 0)
  def drain_all2(k, carry):
    pltpu.make_async_copy(
        rows_v.at[1, :, pl.ds(4 * k, 4), :],
        out_hbm.at[0, pl.ds(base, _RC)], osem1).wait()
    return carry
  lax.fori_loop(0, 50, drain_all2, 0)


_gather = functools.partial(
    pl.kernel,
    out_type=jax.ShapeDtypeStruct((50, _BATCH, 4, _EMBED), jnp.float32),
    mesh=plsc.VectorSubcoreMesh(core_axis_name="c", subcore_axis_name="s"),
    scratch_types=[
        pltpu.VMEM((2, _RC, _HIST), jnp.int32),
        pltpu.VMEM((2, _RC, _HIST, _EMBED), jnp.float32),
        pltpu.SemaphoreType.DMA,
        pltpu.SemaphoreType.DMA,
        pltpu.SemaphoreType.DMA,
    ],
    compiler_params=pltpu.CompilerParams(use_tc_tiling_on_sc=False),
)(_gather_body)


def _retile_body(x_ref, o_ref):
  x3 = jnp.reshape(x_ref[...], (16, 128, 128))
  xt = jnp.transpose(x3, (0, 2, 1))
  o = jnp.reshape(xt, (16, 16, 8, 128))
  o_ref[0] = jnp.transpose(o, (1, 0, 2, 3))


_retile = pl.pallas_call(
    _retile_body,
    out_shape=jax.ShapeDtypeStruct((50, 16, 128, 8, 128), jnp.float32),
    grid=(50, 8),
    in_specs=[pl.BlockSpec((2048, 128), lambda k, bb: (k * 8 + bb, 0))],
    out_specs=pl.BlockSpec(
        (1, 16, 16, 8, 128), lambda k, bb: (k, 0, bb, 0, 0)),
)


def _mask_body(x_ref, o_ref):
  o_ref[...] = x_ref[...] != 0


_mask = pl.pallas_call(
    _mask_body,
    out_shape=jax.ShapeDtypeStruct((_BATCH, _HIST), jnp.bool_),
    grid=(16,),
    in_specs=[pl.BlockSpec((_BATCH // 16, _HIST), lambda i: (i, 0))],
    out_specs=pl.BlockSpec((_BATCH // 16, _HIST), lambda i: (i, 0)),
)


@jax.jit
def kernel(inputs, table):
  rows4 = _gather(inputs, table)
  mask = _mask(inputs)
  out5 = _retile(jnp.reshape(rows4, (50 * _BATCH, 128)))
  emb = jnp.reshape(
      jnp.transpose(out5, (2, 4, 0, 1, 3)), (_BATCH, _HIST, _EMBED))
  return emb, mask


# speedup vs baseline: 3.4457x; 1.0963x over previous
import functools

import jax
import jax.numpy as jnp
from jax import lax
from jax.experimental import pallas as pl
from jax.experimental.pallas import tpu as pltpu
from jax.experimental.pallas import tpu_sc as plsc

_VOCAB = 1000000
_EMBED = 32
_BATCH = 16384
_HIST = 200

_NC = 2
_NS = 16
_NW = _NC * _NS
_RPW = _BATCH // _NW
_RC = 8
_NSTEPS = _RPW // _RC


def _gather_body(idx_hbm, table_hbm, out_hbm, idx_v, rows_v, gsem, osem0,
                 osem1):
  wid = lax.axis_index("s") * _NC + lax.axis_index("c")
  base = wid * _RPW
  osems = (osem0, osem1)

  def do_step(i, b):
    row0 = base + i * _RC
    pltpu.sync_copy(idx_hbm.at[pl.ds(row0, _RC), :], idx_v.at[b])

    @pl.when(i >= 2)
    def _():
      def dr(k, carry):
        pltpu.make_async_copy(
            rows_v.at[b, :, pl.ds(4 * k, 4), :],
            out_hbm.at[0, pl.ds(row0, _RC)], osems[b]).wait()
        return carry
      lax.fori_loop(0, 50, dr, 0)

    for j in range(_RC):
      pltpu.async_copy(table_hbm.at[idx_v.at[b, j]], rows_v.at[b, j], gsem)
    for j in range(_RC):
      pltpu.make_async_copy(
          table_hbm.at[idx_v.at[b, j]], rows_v.at[b, j], gsem).wait()

    def wr(k, carry):
      pltpu.async_copy(
          rows_v.at[b, :, pl.ds(4 * k, 4), :],
          out_hbm.at[k, pl.ds(row0, _RC)], osems[b])
      return carry
    lax.fori_loop(0, 50, wr, 0)

  def pair(g, carry):
    do_step(g * 2, 0)
    do_step(g * 2 + 1, 1)
    return carry

  lax.fori_loop(0, _NSTEPS // 2, pair, 0)
  def drain_all(k, carry):
    pltpu.make_async_copy(
        rows_v.at[0, :, pl.ds(4 * k, 4), :],
        out_hbm.at[0, pl.ds(base, _RC)], osem0).wait()
    return carry
  lax.fori_loop(0, 50, drain_all, 0)
  def drain_all2(k, carry):
    pltpu.make_async_copy(
        rows_v.at[1, :, pl.ds(4 * k, 4), :],
        out_hbm.at[0, pl.ds(base, _RC)], osem1).wait()
    return carry
  lax.fori_loop(0, 50, drain_all2, 0)


_gather = functools.partial(
    pl.kernel,
    out_type=jax.ShapeDtypeStruct((50, _BATCH, 4, _EMBED), jnp.float32),
    mesh=plsc.VectorSubcoreMesh(core_axis_name="c", subcore_axis_name="s"),
    scratch_types=[
        pltpu.VMEM((2, _RC, _HIST), jnp.int32),
        pltpu.VMEM((2, _RC, _HIST, _EMBED), jnp.float32),
        pltpu.SemaphoreType.DMA,
        pltpu.SemaphoreType.DMA,
        pltpu.SemaphoreType.DMA,
    ],
    compiler_params=pltpu.CompilerParams(use_tc_tiling_on_sc=False),
)(_gather_body)


def _retile_body(x_ref, o_ref):
  x3 = jnp.reshape(x_ref[...], (32, 128, 128))
  xt = jnp.transpose(x3, (0, 2, 1))
  o = jnp.reshape(xt, (32, 16, 8, 128))
  o_ref[0] = jnp.transpose(o, (1, 0, 2, 3))


_retile = pl.pallas_call(
    _retile_body,
    out_shape=jax.ShapeDtypeStruct((50, 16, 128, 8, 128), jnp.float32),
    grid=(50, 4),
    in_specs=[pl.BlockSpec((4096, 128), lambda k, bb: (k * 4 + bb, 0))],
    out_specs=pl.BlockSpec(
        (1, 16, 32, 8, 128), lambda k, bb: (k, 0, bb, 0, 0)),
)


def _mask_body(x_ref, o_ref):
  o_ref[...] = x_ref[...] != 0


_mask = pl.pallas_call(
    _mask_body,
    out_shape=jax.ShapeDtypeStruct((_BATCH, _HIST), jnp.bool_),
    grid=(16,),
    in_specs=[pl.BlockSpec((_BATCH // 16, _HIST), lambda i: (i, 0))],
    out_specs=pl.BlockSpec((_BATCH // 16, _HIST), lambda i: (i, 0)),
)


@jax.jit
def kernel(inputs, table):
  rows4 = _gather(inputs, table)
  mask = _mask(inputs)
  out5 = _retile(jnp.reshape(rows4, (50 * _BATCH, 128)))
  emb = jnp.reshape(
      jnp.transpose(out5, (2, 4, 0, 1, 3)), (_BATCH, _HIST, _EMBED))
  return emb, mask


# speedup vs baseline: 3.6063x; 1.0466x over previous
import functools

import jax
import jax.numpy as jnp
from jax import lax
from jax.experimental import pallas as pl
from jax.experimental.pallas import tpu as pltpu
from jax.experimental.pallas import tpu_sc as plsc

_VOCAB = 1000000
_EMBED = 32
_BATCH = 16384
_HIST = 200

_NC = 2
_NS = 16
_NW = _NC * _NS
_RPW = _BATCH // _NW
_RC = 8
_NSTEPS = _RPW // _RC


def _gather_body(idx_hbm, table_hbm, out_hbm, idx_v, rows_v, gsem, osem0,
                 osem1):
  wid = lax.axis_index("s") * _NC + lax.axis_index("c")
  base = wid * _RPW
  osems = (osem0, osem1)

  def do_step(i, b):
    row0 = base + i * _RC
    pltpu.sync_copy(idx_hbm.at[pl.ds(row0, _RC), :], idx_v.at[b])

    @pl.when(i >= 2)
    def _():
      def dr(k, carry):
        pltpu.make_async_copy(
            rows_v.at[b, :, pl.ds(4 * k, 4), :],
            out_hbm.at[0, pl.ds(row0, _RC)], osems[b]).wait()
        return carry
      lax.fori_loop(0, 50, dr, 0)

    for j in range(_RC):
      pltpu.async_copy(table_hbm.at[idx_v.at[b, j]], rows_v.at[b, j], gsem)
    for j in range(_RC):
      pltpu.make_async_copy(
          table_hbm.at[idx_v.at[b, j]], rows_v.at[b, j], gsem).wait()

    def wr(k, carry):
      pltpu.async_copy(
          rows_v.at[b, :, pl.ds(4 * k, 4), :],
          out_hbm.at[k, pl.ds(row0, _RC)], osems[b])
      return carry
    lax.fori_loop(0, 50, wr, 0)

  def pair(g, carry):
    do_step(g * 2, 0)
    do_step(g * 2 + 1, 1)
    return carry

  lax.fori_loop(0, _NSTEPS // 2, pair, 0)
  def drain_all(k, carry):
    pltpu.make_async_copy(
        rows_v.at[0, :, pl.ds(4 * k, 4), :],
        out_hbm.at[0, pl.ds(base, _RC)], osem0).wait()
    return carry
  lax.fori_loop(0, 50, drain_all, 0)
  def drain_all2(k, carry):
    pltpu.make_async_copy(
        rows_v.at[1, :, pl.ds(4 * k, 4), :],
        out_hbm.at[0, pl.ds(base, _RC)], osem1).wait()
    return carry
  lax.fori_loop(0, 50, drain_all2, 0)


_gather = functools.partial(
    pl.kernel,
    out_type=jax.ShapeDtypeStruct((50, _BATCH, 4, _EMBED), jnp.float32),
    mesh=plsc.VectorSubcoreMesh(core_axis_name="c", subcore_axis_name="s"),
    scratch_types=[
        pltpu.VMEM((2, _RC, _HIST), jnp.int32),
        pltpu.VMEM((2, _RC, _HIST, _EMBED), jnp.float32),
        pltpu.SemaphoreType.DMA,
        pltpu.SemaphoreType.DMA,
        pltpu.SemaphoreType.DMA,
    ],
    compiler_params=pltpu.CompilerParams(use_tc_tiling_on_sc=False),
)(_gather_body)


def _retile_body(x_ref, o_ref):
  x3 = jnp.reshape(x_ref[...], (64, 128, 128))
  xt = jnp.transpose(x3, (0, 2, 1))
  o = jnp.reshape(xt, (64, 16, 8, 128))
  o_ref[0] = jnp.transpose(o, (1, 0, 2, 3))


_retile = pl.pallas_call(
    _retile_body,
    out_shape=jax.ShapeDtypeStruct((50, 16, 128, 8, 128), jnp.float32),
    grid=(50, 2),
    in_specs=[pl.BlockSpec((8192, 128), lambda k, bb: (k * 2 + bb, 0))],
    out_specs=pl.BlockSpec(
        (1, 16, 64, 8, 128), lambda k, bb: (k, 0, bb, 0, 0)),
)


def _mask_body(x_ref, o_ref):
  o_ref[...] = x_ref[...] != 0


_mask = pl.pallas_call(
    _mask_body,
    out_shape=jax.ShapeDtypeStruct((_BATCH, _HIST), jnp.bool_),
    grid=(16,),
    in_specs=[pl.BlockSpec((_BATCH // 16, _HIST), lambda i: (i, 0))],
    out_specs=pl.BlockSpec((_BATCH // 16, _HIST), lambda i: (i, 0)),
)


@jax.jit
def kernel(inputs, table):
  rows4 = _gather(inputs, table)
  mask = _mask(inputs)
  out5 = _retile(jnp.reshape(rows4, (50 * _BATCH, 128)))
  emb = jnp.reshape(
      jnp.transpose(out5, (2, 4, 0, 1, 3)), (_BATCH, _HIST, _EMBED))
  return emb, mask
